# baseline (speedup 1.0000x reference)
import functools

import jax
import jax.numpy as jnp
from jax import lax
from jax.experimental import pallas as pl
from jax.experimental.pallas import tpu as pltpu
from jax.experimental.pallas import tpu_sc as plsc

NC = 2
NS = 16
NW = NC * NS
CH = 128


def _mesh():
    return plsc.VectorSubcoreMesh(core_axis_name="c", subcore_axis_name="s",
                                  num_cores=NC, num_subcores=NS)


def _make_deg_lookup(N, Np, E2, D, LCH, K):
    NCH = E2 // CH
    CPW = NCH // NW
    ITERS = CPW // K
    NLC = N // LCH
    RT = Np // NS

    @functools.partial(
        pl.kernel,
        out_type=(jax.ShapeDtypeStruct((NC, Np), jnp.float32),
                  jax.ShapeDtypeStruct((Np, D), jnp.float32)),
        mesh=_mesh(),
        scratch_types=(
            pltpu.VMEM_SHARED((Np,), jnp.float32),
            pltpu.VMEM((RT,), jnp.float32),
            pltpu.VMEM((CH,), jnp.float32),
            pltpu.VMEM((K, 2, CH), jnp.int32),
            pltpu.VMEM((1, LCH), jnp.int32),
            pltpu.VMEM((LCH, D), jnp.float32),
            pltpu.SemaphoreType.DMA,
        ),
    )
    def deg_lookup(x_hbm, sd_hbm, emb_hbm, cnt_out, h_out,
                   cnt_sh, zbuf, ones_v, sdidx, xidx, hrows, sem):
        cid = lax.axis_index("c")
        sid = lax.axis_index("s")
        wid = sid * NC + cid

        z16 = jnp.zeros((16,), jnp.float32)
        for k in range(RT // 16):
            zbuf[pl.ds(k * 16, 16)] = z16
        o16 = jnp.ones((16,), jnp.float32)
        for k in range(CH // 16):
            ones_v[pl.ds(k * 16, 16)] = o16
        pltpu.sync_copy(zbuf, cnt_sh.at[pl.ds(sid * RT, RT)])
        plsc.subcore_barrier()

        def deg_body(it, carry):
            c0 = wid * CPW + it * K
            pltpu.sync_copy(sd_hbm.at[pl.ds(c0, K)], sdidx)
            for j in range(K):
                pltpu.sync_copy(ones_v, cnt_sh.at[sdidx.at[j, 1]], add=True)
            return carry

        lax.fori_loop(0, ITERS, deg_body, 0)

        lo2 = wid * NLC // NW
        hi2 = (wid + 1) * NLC // NW

        def lk_body(k, carry):
            pltpu.sync_copy(x_hbm.at[pl.ds(k * LCH, LCH)], xidx.at[0])
            pltpu.async_copy(emb_hbm.at[xidx.at[0]], hrows, sem).wait()
            pltpu.sync_copy(hrows, h_out.at[pl.ds(k * LCH, LCH)])
            return carry

        lax.fori_loop(lo2, hi2, lk_body, 0)

        plsc.subcore_barrier()
        pltpu.sync_copy(cnt_sh.at[pl.ds(sid * RT, RT)], zbuf)
        pltpu.sync_copy(zbuf, cnt_out.at[cid, pl.ds(sid * RT, RT)])

    return deg_lookup


def _make_prop(Np, E2, FB, NPASS):
    NCH = E2 // CH
    CPW = NCH // NW
    GI = 16
    NB = 4
    GITERS = CPW // GI
    RT = Np // NS
    SUB = RT // CH

    @functools.partial(
        pl.kernel,
        out_type=jax.ShapeDtypeStruct((NC, NPASS, Np, FB), jnp.float32),
        mesh=_mesh(),
        compiler_params=pltpu.CompilerParams(use_tc_tiling_on_sc=False),
        scratch_types=(
            pltpu.VMEM_SHARED((Np, FB), jnp.float32),
            pltpu.VMEM_SHARED((Np, FB), jnp.float32),
            pltpu.VMEM((NB, CH, FB), jnp.float32),
            pltpu.VMEM((GI, 2, CH), jnp.int32),
            pltpu.SemaphoreType.DMA,
            pltpu.SemaphoreType.DMA,
            pltpu.SemaphoreType.DMA,
            pltpu.SemaphoreType.DMA,
        ),
    )
    def prop(sd_hbm, g_hbm, out_hbm, tab_sh, acc_sh, rows, sdidx,
             sem0, sem1, sem2, sem3):
        sems = (sem0, sem1, sem2, sem3)
        cid = lax.axis_index("c")
        sid = lax.axis_index("s")
        wid = cid * NS + sid
        z16 = jnp.zeros((16,), jnp.float32)

        for p in range(NPASS):
            for k in range(SUB):
                pltpu.sync_copy(g_hbm.at[p, pl.ds(sid * RT + k * CH, CH)],
                                rows.at[0])
                pltpu.sync_copy(rows.at[0],
                                tab_sh.at[pl.ds(sid * RT + k * CH, CH)])

            def zb(i, carry):
                for k in range(FB // 16):
                    rows[0, i, pl.ds(k * 16, 16)] = z16
                return carry

            lax.fori_loop(0, CH, zb, 0)
            for k in range(SUB):
                pltpu.sync_copy(rows.at[0],
                                acc_sh.at[pl.ds(sid * RT + k * CH, CH)])
            plsc.subcore_barrier()

            def body(it, carry):
                c0 = wid * CPW + it * GI
                pltpu.sync_copy(sd_hbm.at[pl.ds(c0, GI)], sdidx)
                descs = {}
                for j in range(NB):
                    descs[j] = pltpu.async_copy(tab_sh.at[sdidx.at[j, 0]],
                                                rows.at[j], sems[j])
                for j in range(GI):
                    descs[j].wait()
                    pltpu.sync_copy(rows.at[j % NB],
                                    acc_sh.at[sdidx.at[j, 1]], add=True)
                    if j + NB < GI:
                        descs[j + NB] = pltpu.async_copy(
                            tab_sh.at[sdidx.at[j + NB, 0]],
                            rows.at[(j + NB) % NB], sems[(j + NB) % NB])
                return carry

            lax.fori_loop(0, GITERS, body, 0)

            plsc.subcore_barrier()
            for k in range(SUB):
                pltpu.sync_copy(acc_sh.at[pl.ds(sid * RT + k * CH, CH)],
                                rows.at[0])
                pltpu.sync_copy(rows.at[0],
                                out_hbm.at[cid, p, pl.ds(sid * RT + k * CH, CH)])

    return prop


def _tc1(h, W1, ca, cb, R):
    Np, D = h.shape
    H = W1.shape[1]

    def body(h_ref, w_ref, ca_ref, cb_ref, g_ref):
        dinv = lax.rsqrt(1.0 + ca_ref[...] + cb_ref[...])
        hw = jnp.dot(h_ref[...], w_ref[...], preferred_element_type=jnp.float32)
        g_ref[...] = hw * dinv

    return pl.pallas_call(
        body,
        grid=(Np // R,),
        in_specs=[
            pl.BlockSpec((R, D), lambda i: (i, 0)),
            pl.BlockSpec((D, H), lambda i: (0, 0)),
            pl.BlockSpec((R, 1), lambda i: (i, 0)),
            pl.BlockSpec((R, 1), lambda i: (i, 0)),
        ],
        out_specs=pl.BlockSpec((R, H), lambda i: (i, 0)),
        out_shape=jax.ShapeDtypeStruct((Np, H), jnp.float32),
    )(h, W1, ca, cb)


def _tc2(sa, sb, g1, ca, cb, W2, b1, R):
    Np, H = g1.shape
    OUT = W2.shape[1]

    def body(sa_ref, sb_ref, g1_ref, ca_ref, cb_ref, w_ref, b_ref, g2_ref):
        dinv = lax.rsqrt(1.0 + ca_ref[...] + cb_ref[...])
        s = sa_ref[...] + sb_ref[...] + g1_ref[...]
        h1 = jnp.maximum(dinv * s + b_ref[...], 0.0)
        hw = jnp.dot(h1, w_ref[...], preferred_element_type=jnp.float32)
        g2 = hw * dinv
        g2_ref[0, :, :] = g2[:, :OUT // 2]
        g2_ref[1, :, :] = g2[:, OUT // 2:]

    return pl.pallas_call(
        body,
        grid=(Np // R,),
        in_specs=[
            pl.BlockSpec((R, H), lambda i: (i, 0)),
            pl.BlockSpec((R, H), lambda i: (i, 0)),
            pl.BlockSpec((R, H), lambda i: (i, 0)),
            pl.BlockSpec((R, 1), lambda i: (i, 0)),
            pl.BlockSpec((R, 1), lambda i: (i, 0)),
            pl.BlockSpec((H, OUT), lambda i: (0, 0)),
            pl.BlockSpec((1, H), lambda i: (0, 0)),
        ],
        out_specs=pl.BlockSpec((2, R, OUT // 2), lambda i: (0, i, 0)),
        out_shape=jax.ShapeDtypeStruct((2, Np, OUT // 2), jnp.float32),
    )(sa, sb, g1, ca, cb, W2, b1)


def _tc3(s2r, g2r, ca, cb, b2, R):
    _, NPASS, Np, FB = s2r.shape
    OUT = NPASS * FB

    def body(s_ref, g_ref, ca_ref, cb_ref, b_ref, o_ref):
        dinv = lax.rsqrt(1.0 + ca_ref[...] + cb_ref[...])
        for p in range(NPASS):
            s = s_ref[0, p] + s_ref[1, p] + g_ref[p]
            o_ref[:, pl.ds(p * FB, FB)] = dinv * s + b_ref[:, pl.ds(p * FB, FB)]

    return pl.pallas_call(
        body,
        grid=(Np // R,),
        in_specs=[
            pl.BlockSpec((2, NPASS, R, FB), lambda i: (0, 0, i, 0)),
            pl.BlockSpec((NPASS, R, FB), lambda i: (0, i, 0)),
            pl.BlockSpec((R, 1), lambda i: (i, 0)),
            pl.BlockSpec((R, 1), lambda i: (i, 0)),
            pl.BlockSpec((1, OUT), lambda i: (0, 0)),
        ],
        out_specs=pl.BlockSpec((R, OUT), lambda i: (i, 0)),
        out_shape=jax.ShapeDtypeStruct((Np, OUT), jnp.float32),
    )(s2r, g2r, ca, cb, b2)


def kernel(x, edge_index, emb_table, W1, b1, W2, b2):
    N, D = emb_table.shape
    H = W1.shape[1]
    OUT = W2.shape[1]
    E = edge_index.shape[1]

    tile = NS * CH
    Np = ((N + tile - 1) // tile) * tile
    R = Np // 10
    LCH = max(d for d in range(8, 129, 8) if N % d == 0)

    KD = 8
    KL = 16 * NW * CH
    E2 = ((E + KL - 1) // KL) * KL
    NCH = E2 // CH

    pad_dst = N + jnp.arange(E2 - E, dtype=jnp.int32) % (Np - N)
    src = jnp.concatenate([edge_index[0],
                           jnp.zeros((E2 - E,), jnp.int32)])
    dst = jnp.concatenate([edge_index[1], pad_dst])
    sd = jnp.stack([src.reshape(NCH, CH), dst.reshape(NCH, CH)], axis=1)

    cnt2, h_pad = _make_deg_lookup(N, Np, E2, D, LCH, KD)(x, sd, emb_table)
    ca = cnt2[0].reshape(Np, 1)
    cb = cnt2[1].reshape(Np, 1)

    g1 = _tc1(h_pad, W1, ca, cb, R)
    s1 = _make_prop(Np, E2, H, 1)(sd, g1.reshape(1, Np, H))
    g2 = _tc2(s1[0, 0], s1[1, 0], g1, ca, cb, W2,
              b1.reshape(1, H), R)
    s2 = _make_prop(Np, E2, OUT // 2, 2)(sd, g2)
    out = _tc3(s2, g2, ca, cb, b2.reshape(1, OUT), R)
    return out[:N]


# speedup vs baseline: 1.0227x; 1.0227x over previous
import functools

import jax
import jax.numpy as jnp
from jax import lax
from jax.experimental import pallas as pl
from jax.experimental.pallas import tpu as pltpu
from jax.experimental.pallas import tpu_sc as plsc

NC = 2
NS = 16
NW = NC * NS
CH = 128


def _mesh():
    return plsc.VectorSubcoreMesh(core_axis_name="c", subcore_axis_name="s",
                                  num_cores=NC, num_subcores=NS)


def _make_deg_lookup(N, Np, E2, D, LCH, K):
    NCH = E2 // CH
    CPW = NCH // NW
    ITERS = CPW // K
    NLC = N // LCH
    RT = Np // NS

    @functools.partial(
        pl.kernel,
        out_type=(jax.ShapeDtypeStruct((NC, Np), jnp.float32),
                  jax.ShapeDtypeStruct((Np, D), jnp.float32)),
        mesh=_mesh(),
        scratch_types=(
            pltpu.VMEM_SHARED((Np,), jnp.float32),
            pltpu.VMEM((RT,), jnp.float32),
            pltpu.VMEM((CH,), jnp.float32),
            pltpu.VMEM((K, CH), jnp.int32),
            pltpu.VMEM((1, LCH), jnp.int32),
            pltpu.VMEM((LCH, D), jnp.float32),
            pltpu.SemaphoreType.DMA,
        ),
    )
    def deg_lookup(x_hbm, dst_hbm, emb_hbm, cnt_out, h_out,
                   cnt_sh, zbuf, ones_v, didx, xidx, hrows, sem):
        cid = lax.axis_index("c")
        sid = lax.axis_index("s")
        wid = sid * NC + cid

        z16 = jnp.zeros((16,), jnp.float32)
        for k in range(RT // 16):
            zbuf[pl.ds(k * 16, 16)] = z16
        o16 = jnp.ones((16,), jnp.float32)
        for k in range(CH // 16):
            ones_v[pl.ds(k * 16, 16)] = o16
        pltpu.sync_copy(zbuf, cnt_sh.at[pl.ds(sid * RT, RT)])
        plsc.subcore_barrier()

        def deg_body(it, carry):
            c0 = wid * CPW + it * K
            pltpu.sync_copy(dst_hbm.at[pl.ds(c0, K)], didx)
            for j in range(K):
                pltpu.sync_copy(ones_v, cnt_sh.at[didx.at[j]], add=True)
            return carry

        lax.fori_loop(0, ITERS, deg_body, 0)

        lo2 = wid * NLC // NW
        hi2 = (wid + 1) * NLC // NW

        def lk_body(k, carry):
            pltpu.sync_copy(x_hbm.at[pl.ds(k * LCH, LCH)], xidx.at[0])
            pltpu.async_copy(emb_hbm.at[xidx.at[0]], hrows, sem).wait()
            pltpu.sync_copy(hrows, h_out.at[pl.ds(k * LCH, LCH)])
            return carry

        lax.fori_loop(lo2, hi2, lk_body, 0)

        plsc.subcore_barrier()
        pltpu.sync_copy(cnt_sh.at[pl.ds(sid * RT, RT)], zbuf)
        pltpu.sync_copy(zbuf, cnt_out.at[cid, pl.ds(sid * RT, RT)])

    return deg_lookup


def _make_prop(Np, E2, FB, NPASS):
    NCH = E2 // CH
    CPW = NCH // NW
    GI = 16
    NB = 4
    GITERS = CPW // GI
    RT = Np // NS
    SUB = RT // CH

    @functools.partial(
        pl.kernel,
        out_type=jax.ShapeDtypeStruct((NC, NPASS, Np, FB), jnp.float32),
        mesh=_mesh(),
        compiler_params=pltpu.CompilerParams(use_tc_tiling_on_sc=False),
        scratch_types=(
            pltpu.VMEM_SHARED((Np, FB), jnp.float32),
            pltpu.VMEM_SHARED((Np, FB), jnp.float32),
            pltpu.VMEM((NB, CH, FB), jnp.float32),
            pltpu.VMEM((2, GI, 2, CH), jnp.int32),
            pltpu.SemaphoreType.DMA,
            pltpu.SemaphoreType.DMA,
            pltpu.SemaphoreType.DMA,
            pltpu.SemaphoreType.DMA,
            pltpu.SemaphoreType.DMA,
        ),
    )
    def prop(sd_hbm, g_hbm, out_hbm, tab_sh, acc_sh, rows, sdidx,
             sem0, sem1, sem2, sem3, isem):
        sems = (sem0, sem1, sem2, sem3)
        cid = lax.axis_index("c")
        sid = lax.axis_index("s")
        wid = cid * NS + sid
        z16 = jnp.zeros((16,), jnp.float32)

        for p in range(NPASS):
            for k in range(SUB):
                pltpu.sync_copy(g_hbm.at[p, pl.ds(sid * RT + k * CH, CH)],
                                rows.at[0])
                pltpu.sync_copy(rows.at[0],
                                tab_sh.at[pl.ds(sid * RT + k * CH, CH)])

            def zb(i, carry):
                for k in range(FB // 16):
                    rows[0, i, pl.ds(k * 16, 16)] = z16
                return carry

            lax.fori_loop(0, CH, zb, 0)
            for k in range(SUB):
                pltpu.sync_copy(rows.at[0],
                                acc_sh.at[pl.ds(sid * RT + k * CH, CH)])
            plsc.subcore_barrier()

            pltpu.async_copy(sd_hbm.at[pl.ds(wid * CPW, GI)], sdidx.at[0],
                             isem)

            def body(it2, carry):
                for b in range(2):
                    it = it2 * 2 + b
                    pltpu.make_async_copy(sd_hbm.at[pl.ds(0, GI)],
                                          sdidx.at[b], isem).wait()
                    nxt = wid * CPW + (it + 1) * GI

                    @pl.when(it + 1 < GITERS)
                    def _():
                        pltpu.async_copy(sd_hbm.at[pl.ds(nxt, GI)],
                                         sdidx.at[1 - b], isem)

                    descs = {}
                    for j in range(NB):
                        descs[j] = pltpu.async_copy(
                            tab_sh.at[sdidx.at[b, j, 0]], rows.at[j], sems[j])
                    for j in range(GI):
                        descs[j].wait()
                        pltpu.sync_copy(rows.at[j % NB],
                                        acc_sh.at[sdidx.at[b, j, 1]], add=True)
                        if j + NB < GI:
                            descs[j + NB] = pltpu.async_copy(
                                tab_sh.at[sdidx.at[b, j + NB, 0]],
                                rows.at[(j + NB) % NB], sems[(j + NB) % NB])
                return carry

            lax.fori_loop(0, GITERS // 2, body, 0)

            plsc.subcore_barrier()
            for k in range(SUB):
                pltpu.sync_copy(acc_sh.at[pl.ds(sid * RT + k * CH, CH)],
                                rows.at[0])
                pltpu.sync_copy(rows.at[0],
                                out_hbm.at[cid, p, pl.ds(sid * RT + k * CH, CH)])

    return prop


def _tc1(h, W1, ca, cb, R):
    Np, D = h.shape
    H = W1.shape[1]

    def body(h_ref, w_ref, ca_ref, cb_ref, g_ref):
        dinv = lax.rsqrt(1.0 + ca_ref[...] + cb_ref[...])
        hw = jnp.dot(h_ref[...], w_ref[...], preferred_element_type=jnp.float32)
        g_ref[...] = hw * dinv

    return pl.pallas_call(
        body,
        grid=(Np // R,),
        in_specs=[
            pl.BlockSpec((R, D), lambda i: (i, 0)),
            pl.BlockSpec((D, H), lambda i: (0, 0)),
            pl.BlockSpec((R, 1), lambda i: (i, 0)),
            pl.BlockSpec((R, 1), lambda i: (i, 0)),
        ],
        out_specs=pl.BlockSpec((R, H), lambda i: (i, 0)),
        out_shape=jax.ShapeDtypeStruct((Np, H), jnp.float32),
    )(h, W1, ca, cb)


def _tc2(sa, sb, g1, ca, cb, W2, b1, R):
    Np, H = g1.shape
    OUT = W2.shape[1]

    def body(sa_ref, sb_ref, g1_ref, ca_ref, cb_ref, w_ref, b_ref, g2_ref):
        dinv = lax.rsqrt(1.0 + ca_ref[...] + cb_ref[...])
        s = sa_ref[...] + sb_ref[...] + g1_ref[...]
        h1 = jnp.maximum(dinv * s + b_ref[...], 0.0)
        hw = jnp.dot(h1, w_ref[...], preferred_element_type=jnp.float32)
        g2 = hw * dinv
        g2_ref[0, :, :] = g2[:, :OUT // 2]
        g2_ref[1, :, :] = g2[:, OUT // 2:]

    return pl.pallas_call(
        body,
        grid=(Np // R,),
        in_specs=[
            pl.BlockSpec((R, H), lambda i: (i, 0)),
            pl.BlockSpec((R, H), lambda i: (i, 0)),
            pl.BlockSpec((R, H), lambda i: (i, 0)),
            pl.BlockSpec((R, 1), lambda i: (i, 0)),
            pl.BlockSpec((R, 1), lambda i: (i, 0)),
            pl.BlockSpec((H, OUT), lambda i: (0, 0)),
            pl.BlockSpec((1, H), lambda i: (0, 0)),
        ],
        out_specs=pl.BlockSpec((2, R, OUT // 2), lambda i: (0, i, 0)),
        out_shape=jax.ShapeDtypeStruct((2, Np, OUT // 2), jnp.float32),
    )(sa, sb, g1, ca, cb, W2, b1)


def _tc3(s2r, g2r, ca, cb, b2, R):
    _, NPASS, Np, FB = s2r.shape
    OUT = NPASS * FB

    def body(s_ref, g_ref, ca_ref, cb_ref, b_ref, o_ref):
        dinv = lax.rsqrt(1.0 + ca_ref[...] + cb_ref[...])
        for p in range(NPASS):
            s = s_ref[0, p] + s_ref[1, p] + g_ref[p]
            o_ref[:, pl.ds(p * FB, FB)] = dinv * s + b_ref[:, pl.ds(p * FB, FB)]

    return pl.pallas_call(
        body,
        grid=(Np // R,),
        in_specs=[
            pl.BlockSpec((2, NPASS, R, FB), lambda i: (0, 0, i, 0)),
            pl.BlockSpec((NPASS, R, FB), lambda i: (0, i, 0)),
            pl.BlockSpec((R, 1), lambda i: (i, 0)),
            pl.BlockSpec((R, 1), lambda i: (i, 0)),
            pl.BlockSpec((1, OUT), lambda i: (0, 0)),
        ],
        out_specs=pl.BlockSpec((R, OUT), lambda i: (i, 0)),
        out_shape=jax.ShapeDtypeStruct((Np, OUT), jnp.float32),
    )(s2r, g2r, ca, cb, b2)


def kernel(x, edge_index, emb_table, W1, b1, W2, b2):
    N, D = emb_table.shape
    H = W1.shape[1]
    OUT = W2.shape[1]
    E = edge_index.shape[1]

    tile = NS * CH
    Np = ((N + tile - 1) // tile) * tile
    R = Np // 10
    LCH = max(d for d in range(8, 129, 8) if N % d == 0)

    KD = 8
    KL = 16 * NW * CH
    E2 = ((E + KL - 1) // KL) * KL
    NCH = E2 // CH

    pad_dst = N + jnp.arange(E2 - E, dtype=jnp.int32) % (Np - N)
    src = jnp.concatenate([edge_index[0],
                           jnp.zeros((E2 - E,), jnp.int32)])
    dst2d = jnp.concatenate([edge_index[1], pad_dst]).reshape(NCH, CH)
    sd = jnp.stack([src.reshape(NCH, CH), dst2d], axis=1)

    cnt2, h_pad = _make_deg_lookup(N, Np, E2, D, LCH, KD)(x, dst2d, emb_table)
    ca = cnt2[0].reshape(Np, 1)
    cb = cnt2[1].reshape(Np, 1)

    g1 = _tc1(h_pad, W1, ca, cb, R)
    s1 = _make_prop(Np, E2, H, 1)(sd, g1.reshape(1, Np, H))
    g2 = _tc2(s1[0, 0], s1[1, 0], g1, ca, cb, W2,
              b1.reshape(1, H), R)
    s2 = _make_prop(Np, E2, OUT // 2, 2)(sd, g2)
    out = _tc3(s2, g2, ca, cb, b2.reshape(1, OUT), R)
    return out[:N]


# speedup vs baseline: 1.0641x; 1.0405x over previous
import functools

import jax
import jax.numpy as jnp
from jax import lax
from jax.experimental import pallas as pl
from jax.experimental.pallas import tpu as pltpu
from jax.experimental.pallas import tpu_sc as plsc

NC = 2
NS = 16
NW = NC * NS
CH = 128


def _mesh():
    return plsc.VectorSubcoreMesh(core_axis_name="c", subcore_axis_name="s",
                                  num_cores=NC, num_subcores=NS)


def _make_deg_lookup(N, Np, E2, D, LCH, K):
    NCH = E2 // CH
    CPW = NCH // NW
    ITERS = CPW // K
    NLC = N // LCH
    RT = Np // NS

    @functools.partial(
        pl.kernel,
        out_type=(jax.ShapeDtypeStruct((NC, Np), jnp.float32),
                  jax.ShapeDtypeStruct((Np, D), jnp.float32)),
        mesh=_mesh(),
        scratch_types=(
            pltpu.VMEM_SHARED((Np,), jnp.float32),
            pltpu.VMEM((RT,), jnp.float32),
            pltpu.VMEM((CH,), jnp.float32),
            pltpu.VMEM((K, CH), jnp.int32),
            pltpu.VMEM((1, LCH), jnp.int32),
            pltpu.VMEM((LCH, D), jnp.float32),
            pltpu.SemaphoreType.DMA,
        ),
    )
    def deg_lookup(x_hbm, dst_hbm, emb_hbm, cnt_out, h_out,
                   cnt_sh, zbuf, ones_v, didx, xidx, hrows, sem):
        cid = lax.axis_index("c")
        sid = lax.axis_index("s")
        wid = sid * NC + cid

        z16 = jnp.zeros((16,), jnp.float32)
        for k in range(RT // 16):
            zbuf[pl.ds(k * 16, 16)] = z16
        o16 = jnp.ones((16,), jnp.float32)
        for k in range(CH // 16):
            ones_v[pl.ds(k * 16, 16)] = o16
        pltpu.sync_copy(zbuf, cnt_sh.at[pl.ds(sid * RT, RT)])
        plsc.subcore_barrier()

        def deg_body(it, carry):
            c0 = wid * CPW + it * K
            pltpu.sync_copy(dst_hbm.at[pl.ds(c0, K)], didx)
            for j in range(K):
                pltpu.sync_copy(ones_v, cnt_sh.at[didx.at[j]], add=True)
            return carry

        lax.fori_loop(0, ITERS, deg_body, 0)

        lo2 = wid * NLC // NW
        hi2 = (wid + 1) * NLC // NW

        def lk_body(k, carry):
            pltpu.sync_copy(x_hbm.at[pl.ds(k * LCH, LCH)], xidx.at[0])
            pltpu.async_copy(emb_hbm.at[xidx.at[0]], hrows, sem).wait()
            pltpu.sync_copy(hrows, h_out.at[pl.ds(k * LCH, LCH)])
            return carry

        lax.fori_loop(lo2, hi2, lk_body, 0)

        plsc.subcore_barrier()
        pltpu.sync_copy(cnt_sh.at[pl.ds(sid * RT, RT)], zbuf)
        pltpu.sync_copy(zbuf, cnt_out.at[cid, pl.ds(sid * RT, RT)])

    return deg_lookup


def _make_prop(Np, E2, FB):
    NCH = E2 // CH
    CPT = NCH // NS
    GI = 16
    NB = 4
    GITERS = CPT // GI
    RT = Np // NS
    SUB = RT // CH

    @functools.partial(
        pl.kernel,
        out_type=jax.ShapeDtypeStruct((NC, Np, FB), jnp.float32),
        mesh=_mesh(),
        compiler_params=pltpu.CompilerParams(use_tc_tiling_on_sc=False),
        scratch_types=(
            pltpu.VMEM_SHARED((Np, FB), jnp.float32),
            pltpu.VMEM_SHARED((Np, FB), jnp.float32),
            pltpu.VMEM((NB, CH, FB), jnp.float32),
            pltpu.VMEM((2, GI, 2, CH), jnp.int32),
            pltpu.SemaphoreType.DMA,
            pltpu.SemaphoreType.DMA,
            pltpu.SemaphoreType.DMA,
            pltpu.SemaphoreType.DMA,
            pltpu.SemaphoreType.DMA,
        ),
    )
    def prop(sd_hbm, g_hbm, out_hbm, tab_sh, acc_sh, rows, sdidx,
             sem0, sem1, sem2, sem3, isem):
        sems = (sem0, sem1, sem2, sem3)
        cid = lax.axis_index("c")
        sid = lax.axis_index("s")
        z16 = jnp.zeros((16,), jnp.float32)

        for k in range(SUB):
            pltpu.sync_copy(g_hbm.at[cid, pl.ds(sid * RT + k * CH, CH)],
                            rows.at[0])
            pltpu.sync_copy(rows.at[0],
                            tab_sh.at[pl.ds(sid * RT + k * CH, CH)])

        def zb(i, carry):
            for k in range(FB // 16):
                rows[0, i, pl.ds(k * 16, 16)] = z16
            return carry

        lax.fori_loop(0, CH, zb, 0)
        for k in range(SUB):
            pltpu.sync_copy(rows.at[0],
                            acc_sh.at[pl.ds(sid * RT + k * CH, CH)])
        plsc.subcore_barrier()

        pltpu.async_copy(sd_hbm.at[pl.ds(sid * CPT, GI)], sdidx.at[0], isem)

        def body(it2, carry):
            for b in range(2):
                it = it2 * 2 + b
                pltpu.make_async_copy(sd_hbm.at[pl.ds(0, GI)],
                                      sdidx.at[b], isem).wait()
                nxt = sid * CPT + (it + 1) * GI

                @pl.when(it + 1 < GITERS)
                def _():
                    pltpu.async_copy(sd_hbm.at[pl.ds(nxt, GI)],
                                     sdidx.at[1 - b], isem)

                descs = {}
                for j in range(NB):
                    descs[j] = pltpu.async_copy(
                        tab_sh.at[sdidx.at[b, j, 0]], rows.at[j], sems[j])
                for j in range(GI):
                    descs[j].wait()
                    pltpu.sync_copy(rows.at[j % NB],
                                    acc_sh.at[sdidx.at[b, j, 1]], add=True)
                    if j + NB < GI:
                        descs[j + NB] = pltpu.async_copy(
                            tab_sh.at[sdidx.at[b, j + NB, 0]],
                            rows.at[(j + NB) % NB], sems[(j + NB) % NB])
            return carry

        lax.fori_loop(0, GITERS // 2, body, 0)

        plsc.subcore_barrier()
        for k in range(SUB):
            pltpu.sync_copy(acc_sh.at[pl.ds(sid * RT + k * CH, CH)],
                            rows.at[0])
            pltpu.sync_copy(rows.at[0],
                            out_hbm.at[cid, pl.ds(sid * RT + k * CH, CH)])

    return prop


def _tc1(h, W1, ca, cb, R):
    Np, D = h.shape
    H = W1.shape[1]

    def body(h_ref, w_ref, ca_ref, cb_ref, g_ref):
        dinv = lax.rsqrt(1.0 + ca_ref[...] + cb_ref[...])
        hw = jnp.dot(h_ref[...], w_ref[...], preferred_element_type=jnp.float32)
        g = hw * dinv
        g_ref[0, :, :] = g[:, :H // 2]
        g_ref[1, :, :] = g[:, H // 2:]

    return pl.pallas_call(
        body,
        grid=(Np // R,),
        in_specs=[
            pl.BlockSpec((R, D), lambda i: (i, 0)),
            pl.BlockSpec((D, H), lambda i: (0, 0)),
            pl.BlockSpec((R, 1), lambda i: (i, 0)),
            pl.BlockSpec((R, 1), lambda i: (i, 0)),
        ],
        out_specs=pl.BlockSpec((2, R, H // 2), lambda i: (0, i, 0)),
        out_shape=jax.ShapeDtypeStruct((2, Np, H // 2), jnp.float32),
    )(h, W1, ca, cb)


def _tc2(s1, g1, ca, cb, W2, b1, R):
    _, Np, HH = g1.shape
    H = 2 * HH
    OUT = W2.shape[1]

    def body(s_ref, g1_ref, ca_ref, cb_ref, w_ref, b_ref, g2_ref):
        dinv = lax.rsqrt(1.0 + ca_ref[...] + cb_ref[...])
        s = jnp.concatenate([s_ref[0] + g1_ref[0], s_ref[1] + g1_ref[1]],
                            axis=1)
        h1 = jnp.maximum(dinv * s + b_ref[...], 0.0)
        hw = jnp.dot(h1, w_ref[...], preferred_element_type=jnp.float32)
        g2 = hw * dinv
        g2_ref[0, :, :] = g2[:, :OUT // 2]
        g2_ref[1, :, :] = g2[:, OUT // 2:]

    return pl.pallas_call(
        body,
        grid=(Np // R,),
        in_specs=[
            pl.BlockSpec((2, R, HH), lambda i: (0, i, 0)),
            pl.BlockSpec((2, R, HH), lambda i: (0, i, 0)),
            pl.BlockSpec((R, 1), lambda i: (i, 0)),
            pl.BlockSpec((R, 1), lambda i: (i, 0)),
            pl.BlockSpec((H, OUT), lambda i: (0, 0)),
            pl.BlockSpec((1, H), lambda i: (0, 0)),
        ],
        out_specs=pl.BlockSpec((2, R, OUT // 2), lambda i: (0, i, 0)),
        out_shape=jax.ShapeDtypeStruct((2, Np, OUT // 2), jnp.float32),
    )(s1, g1, ca, cb, W2, b1)


def _tc3(s2, g2r, ca, cb, b2, R):
    _, Np, FB = s2.shape
    OUT = 2 * FB

    def body(s_ref, g_ref, ca_ref, cb_ref, b_ref, o_ref):
        dinv = lax.rsqrt(1.0 + ca_ref[...] + cb_ref[...])
        for p in range(2):
            s = s_ref[p] + g_ref[p]
            o_ref[:, pl.ds(p * FB, FB)] = dinv * s + b_ref[:, pl.ds(p * FB, FB)]

    return pl.pallas_call(
        body,
        grid=(Np // R,),
        in_specs=[
            pl.BlockSpec((2, R, FB), lambda i: (0, i, 0)),
            pl.BlockSpec((2, R, FB), lambda i: (0, i, 0)),
            pl.BlockSpec((R, 1), lambda i: (i, 0)),
            pl.BlockSpec((R, 1), lambda i: (i, 0)),
            pl.BlockSpec((1, OUT), lambda i: (0, 0)),
        ],
        out_specs=pl.BlockSpec((R, OUT), lambda i: (i, 0)),
        out_shape=jax.ShapeDtypeStruct((Np, OUT), jnp.float32),
    )(s2, g2r, ca, cb, b2)


def kernel(x, edge_index, emb_table, W1, b1, W2, b2):
    N, D = emb_table.shape
    H = W1.shape[1]
    OUT = W2.shape[1]
    E = edge_index.shape[1]

    tile = NS * CH
    Np = ((N + tile - 1) // tile) * tile
    R = Np // 10
    LCH = max(d for d in range(8, 129, 8) if N % d == 0)

    KD = 8
    KL = 16 * NW * CH
    E2 = ((E + KL - 1) // KL) * KL
    NCH = E2 // CH

    pad_dst = N + jnp.arange(E2 - E, dtype=jnp.int32) % (Np - N)
    src = jnp.concatenate([edge_index[0],
                           jnp.zeros((E2 - E,), jnp.int32)])
    dst2d = jnp.concatenate([edge_index[1], pad_dst]).reshape(NCH, CH)
    sd = jnp.stack([src.reshape(NCH, CH), dst2d], axis=1)

    cnt2, h_pad = _make_deg_lookup(N, Np, E2, D, LCH, KD)(x, dst2d, emb_table)
    ca = cnt2[0].reshape(Np, 1)
    cb = cnt2[1].reshape(Np, 1)

    g1 = _tc1(h_pad, W1, ca, cb, R)
    s1 = _make_prop(Np, E2, H // 2)(sd, g1)
    g2 = _tc2(s1, g1, ca, cb, W2, b1.reshape(1, H), R)
    s2 = _make_prop(Np, E2, OUT // 2)(sd, g2)
    out = _tc3(s2, g2, ca, cb, b2.reshape(1, OUT), R)
    return out[:N]


# speedup vs baseline: 1.0698x; 1.0054x over previous
import functools

import jax
import jax.numpy as jnp
from jax import lax
from jax.experimental import pallas as pl
from jax.experimental.pallas import tpu as pltpu
from jax.experimental.pallas import tpu_sc as plsc

NC = 2
NS = 16
NW = NC * NS
CH = 128


def _mesh():
    return plsc.VectorSubcoreMesh(core_axis_name="c", subcore_axis_name="s",
                                  num_cores=NC, num_subcores=NS)


def _make_deg_lookup(N, Np, E2, D, LCH, K):
    NCH = E2 // CH
    CPW = NCH // NW
    ITERS = CPW // K
    NLC = N // LCH
    RT = Np // NS

    @functools.partial(
        pl.kernel,
        out_type=(jax.ShapeDtypeStruct((NC, Np), jnp.float32),
                  jax.ShapeDtypeStruct((Np, D), jnp.float32)),
        mesh=_mesh(),
        scratch_types=(
            pltpu.VMEM_SHARED((Np,), jnp.float32),
            pltpu.VMEM((RT,), jnp.float32),
            pltpu.VMEM((CH,), jnp.float32),
            pltpu.VMEM((K, CH), jnp.int32),
            pltpu.VMEM((1, LCH), jnp.int32),
            pltpu.VMEM((LCH, D), jnp.float32),
            pltpu.SemaphoreType.DMA,
        ),
    )
    def deg_lookup(x_hbm, dst_hbm, emb_hbm, cnt_out, h_out,
                   cnt_sh, zbuf, ones_v, didx, xidx, hrows, sem):
        cid = lax.axis_index("c")
        sid = lax.axis_index("s")
        wid = sid * NC + cid

        z16 = jnp.zeros((16,), jnp.float32)
        for k in range(RT // 16):
            zbuf[pl.ds(k * 16, 16)] = z16
        o16 = jnp.ones((16,), jnp.float32)
        for k in range(CH // 16):
            ones_v[pl.ds(k * 16, 16)] = o16
        pltpu.sync_copy(zbuf, cnt_sh.at[pl.ds(sid * RT, RT)])
        plsc.subcore_barrier()

        def deg_body(it, carry):
            c0 = wid * CPW + it * K
            pltpu.sync_copy(dst_hbm.at[pl.ds(c0, K)], didx)
            for j in range(K):
                pltpu.sync_copy(ones_v, cnt_sh.at[didx.at[j]], add=True)
            return carry

        lax.fori_loop(0, ITERS, deg_body, 0)

        lo2 = wid * NLC // NW
        hi2 = (wid + 1) * NLC // NW

        def lk_body(k, carry):
            pltpu.sync_copy(x_hbm.at[pl.ds(k * LCH, LCH)], xidx.at[0])
            pltpu.async_copy(emb_hbm.at[xidx.at[0]], hrows, sem).wait()
            pltpu.sync_copy(hrows, h_out.at[pl.ds(k * LCH, LCH)])
            return carry

        lax.fori_loop(lo2, hi2, lk_body, 0)

        plsc.subcore_barrier()
        pltpu.sync_copy(cnt_sh.at[pl.ds(sid * RT, RT)], zbuf)
        pltpu.sync_copy(zbuf, cnt_out.at[cid, pl.ds(sid * RT, RT)])

    return deg_lookup


def _make_prop(Np, E2, FB):
    NCH = E2 // CH
    CPT = NCH // NS
    GI = 16
    NB = 4
    GITERS = CPT // GI
    RT = Np // NS
    SUB = RT // CH

    @functools.partial(
        pl.kernel,
        out_type=jax.ShapeDtypeStruct((NC, Np, FB), jnp.float32),
        mesh=_mesh(),
        compiler_params=pltpu.CompilerParams(use_tc_tiling_on_sc=False),
        scratch_types=(
            pltpu.VMEM_SHARED((Np, FB), jnp.float32),
            pltpu.VMEM_SHARED((Np, FB), jnp.float32),
            pltpu.VMEM((NB, CH, FB), jnp.float32),
            pltpu.VMEM((2, GI, 2, CH), jnp.int32),
            pltpu.SemaphoreType.DMA,
            pltpu.SemaphoreType.DMA,
            pltpu.SemaphoreType.DMA,
            pltpu.SemaphoreType.DMA,
            pltpu.SemaphoreType.DMA,
        ),
    )
    def prop(sd_hbm, g_hbm, out_hbm, tab_sh, acc_sh, rows, sdidx,
             sem0, sem1, sem2, sem3, isem):
        sems = (sem0, sem1, sem2, sem3)
        cid = lax.axis_index("c")
        sid = lax.axis_index("s")
        z16 = jnp.zeros((16,), jnp.float32)

        for k in range(SUB):
            pltpu.sync_copy(g_hbm.at[cid, pl.ds(sid * RT + k * CH, CH)],
                            rows.at[0])
            pltpu.sync_copy(rows.at[0],
                            tab_sh.at[pl.ds(sid * RT + k * CH, CH)])

        def zb(i, carry):
            for k in range(FB // 16):
                rows[0, i, pl.ds(k * 16, 16)] = z16
            return carry

        lax.fori_loop(0, CH, zb, 0)
        for k in range(SUB):
            pltpu.sync_copy(rows.at[0],
                            acc_sh.at[pl.ds(sid * RT + k * CH, CH)])
        plsc.subcore_barrier()

        pltpu.async_copy(sd_hbm.at[pl.ds(sid * CPT, GI)], sdidx.at[0], isem)

        def body(it2, carry):
            for b in range(2):
                it = it2 * 2 + b
                pltpu.make_async_copy(sd_hbm.at[pl.ds(0, GI)],
                                      sdidx.at[b], isem).wait()
                nxt = sid * CPT + (it + 1) * GI

                @pl.when(it + 1 < GITERS)
                def _():
                    pltpu.async_copy(sd_hbm.at[pl.ds(nxt, GI)],
                                     sdidx.at[1 - b], isem)

                descs = {}
                for j in range(NB):
                    descs[j] = pltpu.async_copy(
                        tab_sh.at[sdidx.at[b, j, 0]], rows.at[j], sems[j])
                for j in range(GI):
                    descs[j].wait()
                    pltpu.sync_copy(rows.at[j % NB],
                                    acc_sh.at[sdidx.at[b, j, 1]], add=True)
                    if j + NB < GI:
                        descs[j + NB] = pltpu.async_copy(
                            tab_sh.at[sdidx.at[b, j + NB, 0]],
                            rows.at[(j + NB) % NB], sems[(j + NB) % NB])
            return carry

        lax.fori_loop(0, GITERS // 2, body, 0)

        plsc.subcore_barrier()
        for k in range(SUB):
            pltpu.sync_copy(acc_sh.at[pl.ds(sid * RT + k * CH, CH)],
                            rows.at[0])
            pltpu.sync_copy(rows.at[0],
                            out_hbm.at[cid, pl.ds(sid * RT + k * CH, CH)])

    return prop


def _tc1(h, W1, ca, cb, R):
    Np, D = h.shape
    H = W1.shape[1]

    def body(h_ref, w_ref, ca_ref, cb_ref, g_ref):
        dinv = lax.rsqrt(1.0 + ca_ref[...] + cb_ref[...])
        hw = jnp.dot(h_ref[...], w_ref[...], preferred_element_type=jnp.float32)
        g = hw * dinv
        g_ref[0, :, :] = g[:, :H // 2]
        g_ref[1, :, :] = g[:, H // 2:]

    return pl.pallas_call(
        body,
        grid=(Np // R,),
        in_specs=[
            pl.BlockSpec((R, D), lambda i: (i, 0)),
            pl.BlockSpec((D, H), lambda i: (0, 0)),
            pl.BlockSpec((R, 1), lambda i: (i, 0)),
            pl.BlockSpec((R, 1), lambda i: (i, 0)),
        ],
        out_specs=pl.BlockSpec((2, R, H // 2), lambda i: (0, i, 0)),
        out_shape=jax.ShapeDtypeStruct((2, Np, H // 2), jnp.float32),
    )(h, W1, ca, cb)


def _tc2(s1, g1, ca, cb, W2, b1, R):
    _, Np, HH = g1.shape
    H = 2 * HH
    OUT = W2.shape[1]

    def body(s_ref, g1_ref, ca_ref, cb_ref, w_ref, b_ref, g2_ref):
        dinv = lax.rsqrt(1.0 + ca_ref[...] + cb_ref[...])
        s = jnp.concatenate([s_ref[0] + g1_ref[0], s_ref[1] + g1_ref[1]],
                            axis=1)
        h1 = jnp.maximum(dinv * s + b_ref[...], 0.0)
        hw = jnp.dot(h1, w_ref[...], preferred_element_type=jnp.float32)
        g2 = hw * dinv
        g2_ref[0, :, :] = g2[:, :OUT // 2]
        g2_ref[1, :, :] = g2[:, OUT // 2:]

    return pl.pallas_call(
        body,
        grid=(Np // R,),
        in_specs=[
            pl.BlockSpec((2, R, HH), lambda i: (0, i, 0)),
            pl.BlockSpec((2, R, HH), lambda i: (0, i, 0)),
            pl.BlockSpec((R, 1), lambda i: (i, 0)),
            pl.BlockSpec((R, 1), lambda i: (i, 0)),
            pl.BlockSpec((H, OUT), lambda i: (0, 0)),
            pl.BlockSpec((1, H), lambda i: (0, 0)),
        ],
        out_specs=pl.BlockSpec((2, R, OUT // 2), lambda i: (0, i, 0)),
        out_shape=jax.ShapeDtypeStruct((2, Np, OUT // 2), jnp.float32),
    )(s1, g1, ca, cb, W2, b1)


def _tc3(s2, g2r, ca, cb, b2, N):
    _, Np, FB = s2.shape
    OUT = 2 * FB
    R3 = N // 10

    def body(s_ref, g_ref, ca_ref, cb_ref, b_ref, o_ref):
        dinv = lax.rsqrt(1.0 + ca_ref[...] + cb_ref[...])
        for p in range(2):
            s = s_ref[p] + g_ref[p]
            o_ref[:, pl.ds(p * FB, FB)] = dinv * s + b_ref[:, pl.ds(p * FB, FB)]

    return pl.pallas_call(
        body,
        grid=(N // R3,),
        in_specs=[
            pl.BlockSpec((2, R3, FB), lambda i: (0, i, 0)),
            pl.BlockSpec((2, R3, FB), lambda i: (0, i, 0)),
            pl.BlockSpec((R3, 1), lambda i: (i, 0)),
            pl.BlockSpec((R3, 1), lambda i: (i, 0)),
            pl.BlockSpec((1, OUT), lambda i: (0, 0)),
        ],
        out_specs=pl.BlockSpec((R3, OUT), lambda i: (i, 0)),
        out_shape=jax.ShapeDtypeStruct((N, OUT), jnp.float32),
    )(s2, g2r, ca, cb, b2)


def kernel(x, edge_index, emb_table, W1, b1, W2, b2):
    N, D = emb_table.shape
    H = W1.shape[1]
    OUT = W2.shape[1]
    E = edge_index.shape[1]

    tile = NS * CH
    Np = ((N + tile - 1) // tile) * tile
    R = Np // 10
    LCH = max(d for d in range(8, 129, 8) if N % d == 0)

    KD = 8
    KL = 16 * NW * CH
    E2 = ((E + KL - 1) // KL) * KL
    NCH = E2 // CH

    pad_dst = N + jnp.arange(E2 - E, dtype=jnp.int32) % (Np - N)
    src = jnp.concatenate([edge_index[0],
                           jnp.zeros((E2 - E,), jnp.int32)])
    dst2d = jnp.concatenate([edge_index[1], pad_dst]).reshape(NCH, CH)
    sd = jnp.stack([src.reshape(NCH, CH), dst2d], axis=1)

    cnt2, h_pad = _make_deg_lookup(N, Np, E2, D, LCH, KD)(x, dst2d, emb_table)
    ca = cnt2[0].reshape(Np, 1)
    cb = cnt2[1].reshape(Np, 1)

    g1 = _tc1(h_pad, W1, ca, cb, R)
    s1 = _make_prop(Np, E2, H // 2)(sd, g1)
    g2 = _tc2(s1, g1, ca, cb, W2, b1.reshape(1, H), R)
    s2 = _make_prop(Np, E2, OUT // 2)(sd, g2)
    return _tc3(s2, g2, ca, cb, b2.reshape(1, OUT), N)
